# baseline (speedup 1.0000x reference)
import jax
import jax.numpy as jnp
from jax.experimental import pallas as pl


def _body(x_ref, e_ref, o_ref):
    o_ref[...] = x_ref[...] + e_ref[...][None, :, :]


def kernel(x, embeddings):
    B, T, D = x.shape
    TS = 256
    return pl.pallas_call(
        _body,
        grid=(T // TS,),
        in_specs=[
            pl.BlockSpec((B, TS, D), lambda i: (0, i, 0)),
            pl.BlockSpec((TS, D), lambda i: (i, 0)),
        ],
        out_specs=pl.BlockSpec((B, TS, D), lambda i: (0, i, 0)),
        out_shape=jax.ShapeDtypeStruct(x.shape, x.dtype),
    )(x, embeddings)


# speedup vs baseline: 1.0006x; 1.0006x over previous
import jax
import jax.numpy as jnp
from jax.experimental import pallas as pl


def _body(x_ref, e_ref, o_ref):
    o_ref[...] = x_ref[...] + e_ref[...][None, :, :]


def kernel(x, embeddings):
    B, T, D = x.shape
    TS = 512
    return pl.pallas_call(
        _body,
        grid=(T // TS,),
        in_specs=[
            pl.BlockSpec((B, TS, D), lambda i: (0, i, 0)),
            pl.BlockSpec((TS, D), lambda i: (i, 0)),
        ],
        out_specs=pl.BlockSpec((B, TS, D), lambda i: (0, i, 0)),
        out_shape=jax.ShapeDtypeStruct(x.shape, x.dtype),
    )(x, embeddings)
